# Initial kernel scaffold; baseline (speedup 1.0000x reference)
import functools

import jax
import jax.numpy as jnp
from jax import lax
from jax.experimental import pallas as pl
from jax.experimental.pallas import tpu as pltpu
from jax.experimental.pallas import tpu_sc as plsc

NU = 25000
NI = 25000
D = 64
NLAYERS = 3
E = 800000

NCORES = 2
NSUB = 16
RPT = 1568
NPAD = NSUB * RPT
PADIDX = NU
CHUNK = 128
CPT = 392
EPT = CPT * CHUNK
EPAD = NSUB * EPT

_MESH = plsc.VectorSubcoreMesh(core_axis_name="c", subcore_axis_name="s")


def _zero_rows(buf, nrows):
    z = jnp.zeros((16,), jnp.float32)

    @pl.loop(0, nrows)
    def _(r):
        for k in range(D // 16):
            buf[r, pl.ds(k * 16, 16)] = z


@functools.partial(
    pl.kernel,
    out_type=jax.ShapeDtypeStruct((2, NPAD), jnp.float32),
    mesh=_MESH,
    scratch_types=[
        pltpu.VMEM((CPT, CHUNK), jnp.int32),
        pltpu.VMEM((NPAD,), jnp.float32),
        pltpu.VMEM((RPT,), jnp.float32),
        pltpu.VMEM((RPT,), jnp.float32),
        pltpu.VMEM_SHARED((NSUB, NPAD), jnp.float32),
    ],
)
def _degree_kernel(sidx_hbm, deg_hbm, idx_v, hist_v, acc_v, tmp_v, hists_sh):
    cid = lax.axis_index("c")
    tid = lax.axis_index("s")
    pltpu.sync_copy(sidx_hbm.at[cid, tid], idx_v)

    z = jnp.zeros((16,), jnp.float32)

    @pl.loop(0, NPAD // 16)
    def _(i):
        hist_v[pl.ds(i * 16, 16)] = z

    ones = jnp.ones((16,), jnp.float32)

    @pl.loop(0, CPT)
    def _(r):
        for k in range(CHUNK // 16):
            idx16 = idx_v[r, pl.ds(k * 16, 16)]
            plsc.addupdate_scatter(hist_v, [idx16], ones)

    pltpu.sync_copy(hist_v, hists_sh.at[tid])
    plsc.subcore_barrier()

    base = tid * RPT

    @pl.loop(0, RPT // 16)
    def _(i):
        acc_v[pl.ds(i * 16, 16)] = z

    for p in range(NSUB):
        pltpu.sync_copy(hists_sh.at[p, pl.ds(base, RPT)], tmp_v)

        @pl.loop(0, RPT // 16)
        def _(i):
            sl = pl.ds(i * 16, 16)
            acc_v[sl] = acc_v[sl] + tmp_v[sl]

    pltpu.sync_copy(acc_v, deg_hbm.at[cid, pl.ds(base, RPT)])


@functools.partial(
    pl.kernel,
    out_type=jax.ShapeDtypeStruct((2, NPAD, D), jnp.float32),
    mesh=_MESH,
    scratch_types=[
        pltpu.VMEM((CPT, CHUNK), jnp.int32),
        pltpu.VMEM((CPT, CHUNK), jnp.int32),
        pltpu.VMEM((CHUNK, D), jnp.float32),
        pltpu.VMEM((CHUNK, D), jnp.float32),
        pltpu.VMEM_SHARED((NPAD, D), jnp.float32),
        pltpu.SemaphoreType.DMA,
        pltpu.SemaphoreType.DMA,
    ],
)
def _layer_kernel(gidx_hbm, sidx_hbm, y_hbm, s_hbm,
                  gidx_v, sidx_v, buf0, buf1, acc_sh, sem0, sem1):
    cid = lax.axis_index("c")
    tid = lax.axis_index("s")
    pltpu.sync_copy(gidx_hbm.at[cid, tid], gidx_v)
    pltpu.sync_copy(sidx_hbm.at[cid, tid], sidx_v)

    _zero_rows(buf0, CHUNK)
    base = tid * RPT
    for r in range(RPT // CHUNK):
        pltpu.sync_copy(buf0, acc_sh.at[pl.ds(base + r * CHUNK, CHUNK)])
    rem = RPT - (RPT // CHUNK) * CHUNK
    if rem:
        pltpu.sync_copy(buf0.at[pl.ds(0, rem)],
                        acc_sh.at[pl.ds(base + (RPT // CHUNK) * CHUNK, rem)])
    plsc.subcore_barrier()

    bufs = (buf0, buf1)
    sems = (sem0, sem1)

    for b in range(2):
        pltpu.async_copy(y_hbm.at[gidx_v.at[b]], bufs[b], sems[b])

    @pl.loop(0, CPT // 2)
    def _(g):
        for b in range(2):
            c = g * 2 + b
            pltpu.make_async_copy(y_hbm.at[gidx_v.at[c]], bufs[b], sems[b]).wait()
            pltpu.sync_copy(bufs[b], acc_sh.at[sidx_v.at[c]], add=True)

            @pl.when(c + 2 < CPT)
            def _():
                pltpu.async_copy(y_hbm.at[gidx_v.at[c + 2]], bufs[b], sems[b])

    plsc.subcore_barrier()
    pltpu.sync_copy(acc_sh.at[pl.ds(base, RPT)],
                    s_hbm.at[cid, pl.ds(base, RPT)])


_RB = 1568


def _dis(deg):
    return jnp.where(deg > 0.0, lax.rsqrt(jnp.maximum(deg, 1.0)), 0.0)


def _scale0_body(x0_ref, deg_ref, y_ref):
    y_ref[...] = x0_ref[...] * _dis(deg_ref[...])


def _scale_body(s_ref, deg_ref, acc_ref, y_ref, accout_ref):
    deg = deg_ref[...]
    s = s_ref[...]
    dis = _dis(deg)
    y_ref[...] = s * (dis * dis)
    accout_ref[...] = acc_ref[...] + s * dis


def _final_body(s_ref, deg_ref, acc_ref, x0_ref, out_ref):
    s = s_ref[...]
    dis = _dis(deg_ref[...])
    out_ref[...] = 0.25 * (x0_ref[...] + acc_ref[...] + s * dis)


def _tc_call(body, n_out, *args):
    grid = (2, NPAD // _RB)
    full = pl.BlockSpec((1, _RB, D), lambda i, j: (i, j, 0))
    degs = pl.BlockSpec((1, _RB, 1), lambda i, j: (i, j, 0))
    specs = [degs if a.shape[-1] == 1 else full for a in args]
    out_sd = jax.ShapeDtypeStruct((2, NPAD, D), jnp.float32)
    return pl.pallas_call(
        body,
        grid=grid,
        in_specs=specs,
        out_specs=[full] * n_out if n_out > 1 else full,
        out_shape=[out_sd] * n_out if n_out > 1 else out_sd,
    )(*args)


def kernel(edge_index, u_emb, i_emb):
    src = edge_index[0]
    dst = edge_index[1]
    pad = jnp.full((EPAD - E,), PADIDX, jnp.int32)
    srcp = jnp.concatenate([src, pad])
    dstp = jnp.concatenate([dst, pad])

    sidx = jnp.stack([srcp, dstp]).reshape(2, NSUB, CPT, CHUNK)
    gidx = jnp.stack([dstp + NPAD, srcp]).reshape(2, NSUB, CPT, CHUNK)

    x0 = jnp.stack([
        jnp.pad(u_emb, ((0, NPAD - NU), (0, 0))),
        jnp.pad(i_emb, ((0, NPAD - NI), (0, 0))),
    ])

    deg = _degree_kernel(sidx)
    deg3 = deg[:, :, None]

    y = _tc_call(_scale0_body, 1, x0, deg3)
    acc = jnp.zeros((2, NPAD, D), jnp.float32)

    out = None
    for layer in range(NLAYERS):
        s = _layer_kernel(gidx, sidx, y.reshape(2 * NPAD, D))
        if layer < NLAYERS - 1:
            y, acc = _tc_call(_scale_body, 2, s, deg3, acc)
        else:
            out = _tc_call(_final_body, 1, s, deg3, acc, x0)

    return out[0, :NU], out[1, :NI]


# speedup vs baseline: 33.4213x; 33.4213x over previous
import functools

import jax
import jax.numpy as jnp
from jax import lax
from jax.experimental import pallas as pl
from jax.experimental.pallas import tpu as pltpu
from jax.experimental.pallas import tpu_sc as plsc

NU = 25000
NI = 25000
D = 64
NLAYERS = 3
E = 800000

NSUB = 16
RPT = 1568
NPAD = NSUB * RPT
PADIDX = NU
CHUNK = 112
CPT = 448
EPT = CPT * CHUNK
EPAD = NSUB * EPT
WCH = 32
NW = RPT // WCH
HCPT = 392

NBUF = 2
NIB = 4
KLAG = 1

_MESH = plsc.VectorSubcoreMesh(core_axis_name="c", subcore_axis_name="s")
_SC_PARAMS = pltpu.CompilerParams(
    use_tc_tiling_on_sc=False, needs_layout_passes=False
)


def _rsqrt_newton(d):
    i = plsc.bitcast(d, jnp.int32)
    r = plsc.bitcast(
        jnp.full((16,), 0x5F3759DF, jnp.int32) - (i >> 1), jnp.float32
    )
    for _ in range(3):
        r = r * (1.5 - 0.5 * d * r * r)
    return r


@functools.partial(
    pl.kernel,
    out_type=(
        jax.ShapeDtypeStruct((2, NSUB, RPT), jnp.float32),
        jax.ShapeDtypeStruct((2, NSUB, RPT, D), jnp.float32),
    ),
    mesh=_MESH,
    scratch_types=[
        pltpu.VMEM((HCPT, 128), jnp.int32),
        pltpu.VMEM((NPAD,), jnp.float32),
        pltpu.VMEM((RPT,), jnp.float32),
        pltpu.VMEM((RPT,), jnp.float32),
        pltpu.VMEM((2, WCH, D), jnp.float32),
        pltpu.VMEM((2, WCH, D), jnp.float32),
        pltpu.VMEM_SHARED((NSUB * NPAD,), jnp.float32),
        pltpu.SemaphoreType.DMA((2,)),
        pltpu.SemaphoreType.DMA((2,)),
    ],
    compiler_params=_SC_PARAMS,
)
def _degree_kernel(sidx_hbm, x0_hbm, deg_hbm, y0_hbm,
                   idx_v, hist_v, dg_v, dis_v, xbuf, ybuf, hists_sh,
                   xsem, ysem):
    cid = lax.axis_index("c")
    tid = lax.axis_index("s")
    pltpu.sync_copy(sidx_hbm.at[cid, tid], idx_v)

    z = jnp.zeros((16,), jnp.float32)

    @pl.loop(0, NPAD // 16)
    def _(i):
        hist_v[pl.ds(i * 16, 16)] = z

    ones = jnp.ones((16,), jnp.float32)

    @pl.loop(0, HCPT)
    def _(r):
        for k in range(128 // 16):
            idx16 = idx_v[r, pl.ds(k * 16, 16)]
            plsc.addupdate_scatter(hist_v, [idx16], ones)

    pltpu.sync_copy(hist_v, hists_sh.at[pl.ds(tid * NPAD, NPAD)])
    plsc.subcore_barrier()

    base = tid * RPT

    @pl.loop(0, RPT // 16)
    def _(i):
        dg_v[pl.ds(i * 16, 16)] = z

    for p in range(NSUB):
        pltpu.sync_copy(hists_sh.at[pl.ds(p * NPAD + base, RPT)], dis_v)

        @pl.loop(0, RPT // 16)
        def _(i):
            sl = pl.ds(i * 16, 16)
            dg_v[sl] = dg_v[sl] + dis_v[sl]

    pltpu.async_copy(dg_v, deg_hbm.at[cid, tid], ysem.at[0])

    @pl.loop(0, RPT // 16)
    def _(i):
        sl = pl.ds(i * 16, 16)
        d = dg_v[sl]
        r = _rsqrt_newton(jnp.maximum(d, 1.0))
        dis_v[sl] = jnp.where(d > 0.0, r, 0.0)

    pltpu.make_async_copy(dg_v, deg_hbm.at[cid, tid], ysem.at[0]).wait()

    @pl.loop(0, NW + 1)
    def _(w):
        b = lax.rem(w, 2)

        @pl.when(w < NW)
        def _():
            pltpu.async_copy(
                x0_hbm.at[cid, pl.ds(base + w * WCH, WCH)], xbuf.at[b],
                xsem.at[b],
            )

        @pl.when(w >= 1)
        def _():
            b1 = lax.rem(w + 1, 2)

            @pl.when(w >= 3)
            def _():
                pltpu.make_async_copy(
                    ybuf.at[b1], y0_hbm.at[cid, tid, pl.ds(0, WCH)],
                    ysem.at[b1],
                ).wait()

            pltpu.make_async_copy(
                x0_hbm.at[cid, pl.ds(base, WCH)], xbuf.at[b1], xsem.at[b1]
            ).wait()
            for g in range(WCH // 16):
                dv = dis_v[pl.ds((w - 1) * WCH + g * 16, 16)]
                for r16 in range(16):
                    r = g * 16 + r16
                    sc = lax.broadcast(dv[r16], (16,))
                    for k in range(D // 16):
                        sl = pl.ds(k * 16, 16)
                        ybuf[b1, r, sl] = xbuf[b1, r, sl] * sc
            pltpu.async_copy(
                ybuf.at[b1], y0_hbm.at[cid, tid, pl.ds((w - 1) * WCH, WCH)],
                ysem.at[b1],
            )

    @pl.loop(0, 2)
    def _(b):
        pltpu.make_async_copy(
            ybuf.at[b], y0_hbm.at[cid, tid, pl.ds(0, WCH)], ysem.at[b]
        ).wait()


@functools.partial(
    pl.kernel,
    out_type=jax.ShapeDtypeStruct((2, NSUB, RPT, D), jnp.float32),
    mesh=_MESH,
    scratch_types=[
        pltpu.VMEM((NIB, CHUNK), jnp.int32),
        pltpu.VMEM((NIB, CHUNK), jnp.int32),
        pltpu.VMEM((NBUF, CHUNK, D), jnp.float32),
        pltpu.VMEM((WCH, D), jnp.float32),
        pltpu.VMEM((2, WCH), jnp.float32),
        pltpu.VMEM((2, WCH, D), jnp.float32),
        pltpu.VMEM_SHARED((NPAD, D), jnp.float32),
        pltpu.SemaphoreType.DMA((NIB,)),
        pltpu.SemaphoreType.DMA((NBUF,)),
        pltpu.SemaphoreType.DMA((NBUF,)),
        pltpu.SemaphoreType.DMA,
        pltpu.SemaphoreType.DMA((2,)),
        pltpu.SemaphoreType.DMA((2,)),
    ],
    compiler_params=_SC_PARAMS,
)
def _layer_kernel(gidx_hbm, sidx_hbm, y_hbm, deg_hbm, yout_hbm,
                  gi_v, si_v, gbuf, zbuf, dbuf, ybuf, acc_sh,
                  isem, gsem, ssem, zsem, dsem, ysem):
    cid = lax.axis_index("c")
    tid = lax.axis_index("s")
    ebase = (cid * NSUB + tid) * EPT
    base = tid * RPT

    z = jnp.zeros((16,), jnp.float32)

    @pl.loop(0, WCH)
    def _(r):
        for k in range(D // 16):
            zbuf[r, pl.ds(k * 16, 16)] = z

    @pl.loop(0, NW)
    def _(w):
        pltpu.async_copy(zbuf, acc_sh.at[pl.ds(base + w * WCH, WCH)], zsem)

    @pl.loop(0, NW)
    def _(w):
        pltpu.make_async_copy(zbuf, acc_sh.at[pl.ds(base, WCH)], zsem).wait()

    plsc.subcore_barrier()

    GLAG = 2
    DLAG = NBUF - KLAG
    TLAG = GLAG + KLAG + DLAG

    @pl.loop(0, CPT + TLAG)
    def _(c):
        @pl.when(c >= TLAG)
        def _():
            bd = lax.rem(c + NBUF - KLAG - DLAG, NBUF)
            pltpu.make_async_copy(
                gbuf.at[bd], acc_sh.at[pl.ds(base, CHUNK)], ssem.at[bd]
            ).wait()

        @pl.when(c < CPT)
        def _():
            bf = lax.rem(c, NIB)
            eoff = ebase + c * CHUNK
            pltpu.async_copy(
                gidx_hbm.at[pl.ds(eoff, CHUNK)], gi_v.at[bf], isem.at[bf]
            )
            pltpu.async_copy(
                sidx_hbm.at[pl.ds(eoff, CHUNK)], si_v.at[bf], isem.at[bf]
            )

        @pl.when((c >= GLAG) & (c < CPT + GLAG))
        def _():
            bg = lax.rem(c + NIB - GLAG, NIB)
            b1 = lax.rem(c + NBUF - GLAG, NBUF)
            pltpu.make_async_copy(
                gidx_hbm.at[pl.ds(ebase, CHUNK)], gi_v.at[bg], isem.at[bg]
            ).wait()
            pltpu.make_async_copy(
                sidx_hbm.at[pl.ds(ebase, CHUNK)], si_v.at[bg], isem.at[bg]
            ).wait()
            pltpu.async_copy(
                y_hbm.at[gi_v.at[bg]], gbuf.at[b1], gsem.at[b1]
            )

        @pl.when((c >= GLAG + KLAG) & (c < CPT + GLAG + KLAG))
        def _():
            bs = lax.rem(c + NIB - GLAG - KLAG, NIB)
            b2 = lax.rem(c + NBUF - GLAG - KLAG, NBUF)
            pltpu.make_async_copy(
                y_hbm.at[gi_v.at[bs]], gbuf.at[b2], gsem.at[b2]
            ).wait()
            pltpu.async_copy(
                gbuf.at[b2], acc_sh.at[si_v.at[bs]], ssem.at[b2], add=True
            )

    plsc.subcore_barrier()

    @pl.loop(0, NW + 1)
    def _(w):
        b = lax.rem(w, 2)

        @pl.when(w < NW)
        def _():
            pltpu.async_copy(
                deg_hbm.at[cid, tid, pl.ds(w * WCH, WCH)], dbuf.at[b],
                dsem.at[b],
            )

        @pl.when(w >= 1)
        def _():
            b1 = lax.rem(w + 1, 2)

            @pl.when(w >= 3)
            def _():
                pltpu.make_async_copy(
                    ybuf.at[b1], yout_hbm.at[cid, tid, pl.ds(0, WCH)],
                    ysem.at[b1],
                ).wait()

            pltpu.sync_copy(
                acc_sh.at[pl.ds(base + (w - 1) * WCH, WCH)], ybuf.at[b1]
            )
            pltpu.make_async_copy(
                deg_hbm.at[cid, tid, pl.ds(0, WCH)], dbuf.at[b1],
                dsem.at[b1],
            ).wait()
            for g in range(WCH // 16):
                sl = pl.ds(g * 16, 16)
                d = dbuf[b1, sl]
                rec = 1.0 / jnp.maximum(d, 1.0)
                dv = jnp.where(d > 0.0, rec, 0.0)
                for r16 in range(16):
                    r = g * 16 + r16
                    sc = lax.broadcast(dv[r16], (16,))
                    for k in range(D // 16):
                        sl2 = pl.ds(k * 16, 16)
                        ybuf[b1, r, sl2] = ybuf[b1, r, sl2] * sc
            pltpu.async_copy(
                ybuf.at[b1], yout_hbm.at[cid, tid, pl.ds((w - 1) * WCH, WCH)],
                ysem.at[b1],
            )

    @pl.loop(0, 2)
    def _(b):
        pltpu.make_async_copy(
            ybuf.at[b], yout_hbm.at[cid, tid, pl.ds(0, WCH)], ysem.at[b]
        ).wait()


_RB = 1568


def _final_body(x0_ref, deg_ref, y1_ref, y2_ref, y3_ref, out_ref):
    rootd = jnp.sqrt(deg_ref[...])
    ysum = y1_ref[...] + y2_ref[...] + y3_ref[...]
    out_ref[...] = 0.25 * (x0_ref[...] + rootd * ysum)


def _final_tc(x0, deg3, y1, y2, y3):
    grid = (2, NPAD // _RB)
    full = pl.BlockSpec((1, _RB, D), lambda i, j: (i, j, 0))
    degs = pl.BlockSpec((1, _RB, 1), lambda i, j: (i, j, 0))
    return pl.pallas_call(
        _final_body,
        grid=grid,
        in_specs=[full, degs, full, full, full],
        out_specs=full,
        out_shape=jax.ShapeDtypeStruct((2, NPAD, D), jnp.float32),
    )(x0, deg3, y1, y2, y3)


def kernel(edge_index, u_emb, i_emb):
    src = edge_index[0]
    dst = edge_index[1]
    pad = jnp.full((EPAD - E,), PADIDX, jnp.int32)
    srcp = jnp.concatenate([src, pad])
    dstp = jnp.concatenate([dst, pad])

    sidx = jnp.concatenate([srcp, dstp])
    sidx4 = sidx.reshape(2, NSUB, HCPT, 128)
    gidx = jnp.concatenate([dstp + NPAD, srcp])

    x0 = jnp.stack([
        jnp.pad(u_emb, ((0, NPAD - NU), (0, 0))),
        jnp.pad(i_emb, ((0, NPAD - NI), (0, 0))),
    ])

    deg, y = _degree_kernel(sidx4, x0)

    ys = []
    for _ in range(NLAYERS):
        y = _layer_kernel(gidx, sidx, y.reshape(2 * NPAD, D), deg)
        ys.append(y.reshape(2, NPAD, D))

    deg3 = deg.reshape(2, NPAD)[:, :, None]
    out = _final_tc(x0, deg3, *ys)
    return out[0, :NU], out[1, :NI]


# speedup vs baseline: 34.4168x; 1.0298x over previous
import functools

import jax
import jax.numpy as jnp
from jax import lax
from jax.experimental import pallas as pl
from jax.experimental.pallas import tpu as pltpu
from jax.experimental.pallas import tpu_sc as plsc

NU = 25000
NI = 25000
D = 64
NLAYERS = 3
E = 800000

NSUB = 16
RPT = 1568
NPAD = NSUB * RPT
PADIDX = NU
CHUNK = 128
CPT = 392
EPT = CPT * CHUNK
EPAD = NSUB * EPT
WCH = 32
NW = RPT // WCH
HCPT = 392

NBUF = 2
NIB = 4
KLAG = 1

_MESH = plsc.VectorSubcoreMesh(core_axis_name="c", subcore_axis_name="s")
_SC_PARAMS = pltpu.CompilerParams(
    use_tc_tiling_on_sc=False, needs_layout_passes=False
)


def _rsqrt_newton(d):
    i = plsc.bitcast(d, jnp.int32)
    r = plsc.bitcast(
        jnp.full((16,), 0x5F3759DF, jnp.int32) - (i >> 1), jnp.float32
    )
    for _ in range(3):
        r = r * (1.5 - 0.5 * d * r * r)
    return r


@functools.partial(
    pl.kernel,
    out_type=(
        jax.ShapeDtypeStruct((2, NSUB, RPT), jnp.float32),
        jax.ShapeDtypeStruct((2, NSUB, RPT, D), jnp.float32),
    ),
    mesh=_MESH,
    scratch_types=[
        pltpu.VMEM((HCPT, 128), jnp.int32),
        pltpu.VMEM((NPAD,), jnp.float32),
        pltpu.VMEM((RPT,), jnp.float32),
        pltpu.VMEM((RPT,), jnp.float32),
        pltpu.VMEM((2, WCH, D), jnp.float32),
        pltpu.VMEM((2, WCH, D), jnp.float32),
        pltpu.VMEM_SHARED((NSUB * NPAD,), jnp.float32),
        pltpu.SemaphoreType.DMA((2,)),
        pltpu.SemaphoreType.DMA((2,)),
    ],
    compiler_params=_SC_PARAMS,
)
def _degree_kernel(sidx_hbm, x0_hbm, deg_hbm, y0_hbm,
                   idx_v, hist_v, dg_v, dis_v, xbuf, ybuf, hists_sh,
                   xsem, ysem):
    cid = lax.axis_index("c")
    tid = lax.axis_index("s")
    pltpu.sync_copy(sidx_hbm.at[cid, tid], idx_v)

    z = jnp.zeros((16,), jnp.float32)

    @pl.loop(0, NPAD // 16)
    def _(i):
        hist_v[pl.ds(i * 16, 16)] = z

    ones = jnp.ones((16,), jnp.float32)

    @pl.loop(0, HCPT)
    def _(r):
        for k in range(128 // 16):
            idx16 = idx_v[r, pl.ds(k * 16, 16)]
            plsc.addupdate_scatter(hist_v, [idx16], ones)

    pltpu.sync_copy(hist_v, hists_sh.at[pl.ds(tid * NPAD, NPAD)])
    plsc.subcore_barrier()

    base = tid * RPT

    @pl.loop(0, RPT // 16)
    def _(i):
        dg_v[pl.ds(i * 16, 16)] = z

    for p in range(NSUB):
        pltpu.sync_copy(hists_sh.at[pl.ds(p * NPAD + base, RPT)], dis_v)

        @pl.loop(0, RPT // 16)
        def _(i):
            sl = pl.ds(i * 16, 16)
            dg_v[sl] = dg_v[sl] + dis_v[sl]

    pltpu.async_copy(dg_v, deg_hbm.at[cid, tid], ysem.at[0])

    @pl.loop(0, RPT // 16)
    def _(i):
        sl = pl.ds(i * 16, 16)
        d = dg_v[sl]
        r = _rsqrt_newton(jnp.maximum(d, 1.0))
        dis_v[sl] = jnp.where(d > 0.0, r, 0.0)

    pltpu.make_async_copy(dg_v, deg_hbm.at[cid, tid], ysem.at[0]).wait()

    @pl.loop(0, NW + 1)
    def _(w):
        b = lax.rem(w, 2)

        @pl.when(w < NW)
        def _():
            pltpu.async_copy(
                x0_hbm.at[cid, pl.ds(base + w * WCH, WCH)], xbuf.at[b],
                xsem.at[b],
            )

        @pl.when(w >= 1)
        def _():
            b1 = lax.rem(w + 1, 2)

            @pl.when(w >= 3)
            def _():
                pltpu.make_async_copy(
                    ybuf.at[b1], y0_hbm.at[cid, tid, pl.ds(0, WCH)],
                    ysem.at[b1],
                ).wait()

            pltpu.make_async_copy(
                x0_hbm.at[cid, pl.ds(base, WCH)], xbuf.at[b1], xsem.at[b1]
            ).wait()
            for g in range(WCH // 16):
                dv = dis_v[pl.ds((w - 1) * WCH + g * 16, 16)]
                for r16 in range(16):
                    r = g * 16 + r16
                    sc = lax.broadcast(dv[r16], (16,))
                    for k in range(D // 16):
                        sl = pl.ds(k * 16, 16)
                        ybuf[b1, r, sl] = xbuf[b1, r, sl] * sc
            pltpu.async_copy(
                ybuf.at[b1], y0_hbm.at[cid, tid, pl.ds((w - 1) * WCH, WCH)],
                ysem.at[b1],
            )

    @pl.loop(0, 2)
    def _(b):
        pltpu.make_async_copy(
            ybuf.at[b], y0_hbm.at[cid, tid, pl.ds(0, WCH)], ysem.at[b]
        ).wait()


@functools.partial(
    pl.kernel,
    out_type=jax.ShapeDtypeStruct((2, NSUB, RPT, D), jnp.float32),
    mesh=_MESH,
    scratch_types=[
        pltpu.VMEM((NIB, CHUNK), jnp.int32),
        pltpu.VMEM((NIB, CHUNK), jnp.int32),
        pltpu.VMEM((NBUF, CHUNK, D), jnp.float32),
        pltpu.VMEM((WCH, D), jnp.float32),
        pltpu.VMEM((2, WCH), jnp.float32),
        pltpu.VMEM((2, WCH, D), jnp.float32),
        pltpu.VMEM_SHARED((NPAD, D), jnp.float32),
        pltpu.SemaphoreType.DMA((NIB,)),
        pltpu.SemaphoreType.DMA((NBUF,)),
        pltpu.SemaphoreType.DMA((NBUF,)),
        pltpu.SemaphoreType.DMA,
        pltpu.SemaphoreType.DMA((2,)),
        pltpu.SemaphoreType.DMA((2,)),
    ],
    compiler_params=_SC_PARAMS,
)
def _layer_kernel(gidx_hbm, sidx_hbm, y_hbm, deg_hbm, yout_hbm,
                  gi_v, si_v, gbuf, zbuf, dbuf, ybuf, acc_sh,
                  isem, gsem, ssem, zsem, dsem, ysem):
    cid = lax.axis_index("c")
    tid = lax.axis_index("s")
    ebase = (cid * NSUB + tid) * EPT
    base = tid * RPT

    z = jnp.zeros((16,), jnp.float32)

    @pl.loop(0, WCH)
    def _(r):
        for k in range(D // 16):
            zbuf[r, pl.ds(k * 16, 16)] = z

    @pl.loop(0, NW)
    def _(w):
        pltpu.async_copy(zbuf, acc_sh.at[pl.ds(base + w * WCH, WCH)], zsem)

    @pl.loop(0, NW)
    def _(w):
        pltpu.make_async_copy(zbuf, acc_sh.at[pl.ds(base, WCH)], zsem).wait()

    plsc.subcore_barrier()

    GLAG = 2
    DLAG = NBUF - KLAG
    TLAG = GLAG + KLAG + DLAG

    @pl.loop(0, CPT + TLAG)
    def _(c):
        @pl.when(c >= TLAG)
        def _():
            bd = lax.rem(c + NBUF - KLAG - DLAG, NBUF)
            pltpu.make_async_copy(
                gbuf.at[bd], acc_sh.at[pl.ds(base, CHUNK)], ssem.at[bd]
            ).wait()

        @pl.when(c < CPT)
        def _():
            bf = lax.rem(c, NIB)
            eoff = ebase + c * CHUNK
            pltpu.async_copy(
                gidx_hbm.at[pl.ds(eoff, CHUNK)], gi_v.at[bf], isem.at[bf]
            )
            pltpu.async_copy(
                sidx_hbm.at[pl.ds(eoff, CHUNK)], si_v.at[bf], isem.at[bf]
            )

        @pl.when((c >= GLAG) & (c < CPT + GLAG))
        def _():
            bg = lax.rem(c + NIB - GLAG, NIB)
            b1 = lax.rem(c + NBUF - GLAG, NBUF)
            pltpu.make_async_copy(
                gidx_hbm.at[pl.ds(ebase, CHUNK)], gi_v.at[bg], isem.at[bg]
            ).wait()
            pltpu.make_async_copy(
                sidx_hbm.at[pl.ds(ebase, CHUNK)], si_v.at[bg], isem.at[bg]
            ).wait()
            pltpu.async_copy(
                y_hbm.at[gi_v.at[bg]], gbuf.at[b1], gsem.at[b1]
            )

        @pl.when((c >= GLAG + KLAG) & (c < CPT + GLAG + KLAG))
        def _():
            bs = lax.rem(c + NIB - GLAG - KLAG, NIB)
            b2 = lax.rem(c + NBUF - GLAG - KLAG, NBUF)
            pltpu.make_async_copy(
                y_hbm.at[gi_v.at[bs]], gbuf.at[b2], gsem.at[b2]
            ).wait()
            pltpu.async_copy(
                gbuf.at[b2], acc_sh.at[si_v.at[bs]], ssem.at[b2], add=True
            )

    plsc.subcore_barrier()

    @pl.loop(0, NW + 1)
    def _(w):
        b = lax.rem(w, 2)

        @pl.when(w < NW)
        def _():
            pltpu.async_copy(
                deg_hbm.at[cid, tid, pl.ds(w * WCH, WCH)], dbuf.at[b],
                dsem.at[b],
            )

        @pl.when(w >= 1)
        def _():
            b1 = lax.rem(w + 1, 2)

            @pl.when(w >= 3)
            def _():
                pltpu.make_async_copy(
                    ybuf.at[b1], yout_hbm.at[cid, tid, pl.ds(0, WCH)],
                    ysem.at[b1],
                ).wait()

            pltpu.sync_copy(
                acc_sh.at[pl.ds(base + (w - 1) * WCH, WCH)], ybuf.at[b1]
            )
            pltpu.make_async_copy(
                deg_hbm.at[cid, tid, pl.ds(0, WCH)], dbuf.at[b1],
                dsem.at[b1],
            ).wait()
            for g in range(WCH // 16):
                sl = pl.ds(g * 16, 16)
                d = dbuf[b1, sl]
                rec = 1.0 / jnp.maximum(d, 1.0)
                dv = jnp.where(d > 0.0, rec, 0.0)
                for r16 in range(16):
                    r = g * 16 + r16
                    sc = lax.broadcast(dv[r16], (16,))
                    for k in range(D // 16):
                        sl2 = pl.ds(k * 16, 16)
                        ybuf[b1, r, sl2] = ybuf[b1, r, sl2] * sc
            pltpu.async_copy(
                ybuf.at[b1], yout_hbm.at[cid, tid, pl.ds((w - 1) * WCH, WCH)],
                ysem.at[b1],
            )

    @pl.loop(0, 2)
    def _(b):
        pltpu.make_async_copy(
            ybuf.at[b], yout_hbm.at[cid, tid, pl.ds(0, WCH)], ysem.at[b]
        ).wait()


_RB = 1568


def _final_body(x0_ref, deg_ref, y1_ref, y2_ref, y3_ref, out_ref):
    rootd = jnp.sqrt(deg_ref[...])
    ysum = y1_ref[...] + y2_ref[...] + y3_ref[...]
    out_ref[...] = 0.25 * (x0_ref[...] + rootd * ysum)


def _final_tc(x0, deg3, y1, y2, y3):
    grid = (2, NPAD // _RB)
    full = pl.BlockSpec((1, _RB, D), lambda i, j: (i, j, 0))
    degs = pl.BlockSpec((1, _RB, 1), lambda i, j: (i, j, 0))
    return pl.pallas_call(
        _final_body,
        grid=grid,
        in_specs=[full, degs, full, full, full],
        out_specs=full,
        out_shape=jax.ShapeDtypeStruct((2, NPAD, D), jnp.float32),
    )(x0, deg3, y1, y2, y3)


def kernel(edge_index, u_emb, i_emb):
    src = edge_index[0]
    dst = edge_index[1]
    pad = jnp.full((EPAD - E,), PADIDX, jnp.int32)
    srcp = jnp.concatenate([src, pad])
    dstp = jnp.concatenate([dst, pad])

    sidx = jnp.concatenate([srcp, dstp])
    sidx4 = sidx.reshape(2, NSUB, HCPT, 128)
    gidx = jnp.concatenate([dstp + NPAD, srcp])

    x0 = jnp.stack([
        jnp.pad(u_emb, ((0, NPAD - NU), (0, 0))),
        jnp.pad(i_emb, ((0, NPAD - NI), (0, 0))),
    ])

    deg, y = _degree_kernel(sidx4, x0)

    ys = []
    for _ in range(NLAYERS):
        y = _layer_kernel(gidx, sidx, y.reshape(2 * NPAD, D), deg)
        ys.append(y.reshape(2, NPAD, D))

    deg3 = deg.reshape(2, NPAD)[:, :, None]
    out = _final_tc(x0, deg3, *ys)
    return out[0, :NU], out[1, :NI]
